# baseline (speedup 1.0000x reference)
import functools

import jax
import jax.numpy as jnp
from jax import lax
from jax.experimental import pallas as pl
from jax.experimental.pallas import tpu as pltpu
from jax.experimental.pallas import tpu_sc as plsc

_NC = 2
_NS = 16
_NW = _NC * _NS
_CB = 128
_L = 16
_VB = 128


def _diag_ids():
    rd = lax.iota(jnp.int32, _L)
    cds = [lax.rem(lax.iota(jnp.int32, _L) + s, _L) for s in range(_L)]
    return rd, cds


@functools.lru_cache(maxsize=None)
def _transpose_call(v_dim, d):
    v_pad = ((v_dim + _VB - 1) // _VB) * _VB
    full_blocks = v_dim // _VB
    per_w = full_blocks // _NW
    extra = full_blocks - per_w * _NW
    pairs = per_w // 2
    mesh = plsc.VectorSubcoreMesh(core_axis_name="c", subcore_axis_name="s")

    @functools.partial(
        pl.kernel,
        mesh=mesh,
        out_type=jax.ShapeDtypeStruct((v_pad, 2 * d), jnp.float32),
        compiler_params=pltpu.CompilerParams(use_tc_tiling_on_sc=True, needs_layout_passes=False),
        scratch_types=[
            pltpu.VMEM((d, _VB), jnp.float32),
            pltpu.VMEM((d, _VB), jnp.float32),
            pltpu.VMEM((_VB, 2 * d), jnp.float32),
            pltpu.VMEM((_VB, 2 * d), jnp.float32),
            pltpu.SemaphoreType.DMA,
            pltpu.SemaphoreType.DMA,
            pltpu.SemaphoreType.DMA,
            pltpu.SemaphoreType.DMA,
        ],
    )
    def k1(wt_hbm, tail_hbm, scr_hbm, sb0, sb1, db0, db1,
           rs0, rs1, ws0, ws1):
        wid = lax.axis_index("s") * _NC + lax.axis_index("c")
        u0 = wid * per_w
        sbs = (sb0, sb1)
        dbs = (db0, db1)
        rsems = (rs0, rs1)
        wsems = (ws0, ws1)
        rd, cds = _diag_ids()

        def fire_read(b, u):
            pltpu.async_copy(
                wt_hbm.at[:, pl.ds((u0 + u) * _VB, _VB)], sbs[b], rsems[b])

        def wait_read(b):
            pltpu.make_async_copy(
                wt_hbm.at[:, pl.ds(0, _VB)], sbs[b], rsems[b]).wait()

        def transpose(src, dst):
            @plsc.parallel_loop(0, _VB // _L)
            def vblk(vg):
                v0 = vg * _L
                for cg in range(d // _L):
                    cvec = rd + (cg * _L)
                    for s in range(_L):
                        vvec = cds[s] + v0
                        val = plsc.load_gather(src, [cvec, vvec])
                        plsc.store_scatter(dst, [vvec, cvec], val)

        def scr_slice(u):
            return scr_hbm.at[pl.ds((u0 + u) * _VB, _VB), :]

        def fire_store(b, u):
            pltpu.make_async_copy(dbs[b], scr_slice(u), wsems[b]).start()

        def wait_store(b, u):
            pltpu.make_async_copy(dbs[b], scr_slice(u), wsems[b]).wait()

        fire_read(0, 0)
        fire_read(1, 1)

        def body(p, carry):
            for b in range(2):
                u = 2 * p + b
                wait_read(b)

                @pl.when(p > 0)
                def _():
                    wait_store(b, u)

                transpose(sbs[b], dbs[b])
                fire_store(b, u)

                @pl.when(u + 2 < per_w)
                def _():
                    fire_read(b, u + 2)

            return carry

        lax.fori_loop(0, pairs, body, 0)
        wait_store(0, per_w - 2)
        wait_store(1, per_w - 1)

        @pl.when(wid < extra)
        def _():
            ub = full_blocks - extra + wid
            pltpu.sync_copy(wt_hbm.at[:, pl.ds(ub * _VB, _VB)], sb0)
            transpose(sb0, db0)
            pltpu.sync_copy(db0, scr_hbm.at[pl.ds(ub * _VB, _VB), :])

        @pl.when(wid == extra)
        def _():
            pltpu.sync_copy(tail_hbm, sb1)
            transpose(sb1, db1)
            pltpu.sync_copy(db1, scr_hbm.at[pl.ds(full_blocks * _VB, _VB), :])

    return k1


@functools.lru_cache(maxsize=None)
def _gather_call(t_dim, b_dim, v_pad, d):
    chunks_per_t = b_dim // _CB
    chunks_total = t_dim * chunks_per_t
    chunks_per_w = chunks_total // _NW
    pairs = chunks_per_w // 2
    mesh = plsc.VectorSubcoreMesh(core_axis_name="c", subcore_axis_name="s")

    @functools.partial(
        pl.kernel,
        mesh=mesh,
        out_type=jax.ShapeDtypeStruct((t_dim, d, b_dim), jnp.float32),
        compiler_params=pltpu.CompilerParams(use_tc_tiling_on_sc=True, needs_layout_passes=False),
        scratch_types=[
            pltpu.VMEM((chunks_per_w, _CB), jnp.int32),
            pltpu.VMEM((_CB, 2 * d), jnp.float32),
            pltpu.VMEM((_CB, 2 * d), jnp.float32),
            pltpu.VMEM((d, _CB), jnp.float32),
            pltpu.VMEM((d, _CB), jnp.float32),
            pltpu.SemaphoreType.DMA,
            pltpu.SemaphoreType.DMA,
            pltpu.SemaphoreType.DMA,
            pltpu.SemaphoreType.DMA,
        ],
    )
    def k2(idx_hbm, scr_hbm, out_hbm, idx_v, rows0, rows1, tb0, tb1,
           gsem0, gsem1, ssem0, ssem1):
        wid = lax.axis_index("s") * _NC + lax.axis_index("c")
        c0 = wid * chunks_per_w
        pltpu.sync_copy(idx_hbm.at[pl.ds(c0, chunks_per_w)], idx_v)
        rows = (rows0, rows1)
        tbs = (tb0, tb1)
        gsems = (gsem0, gsem1)
        ssems = (ssem0, ssem1)
        rd, cds = _diag_ids()

        def out_slice(u):
            ug = c0 + u
            t = ug // chunks_per_t
            cb = lax.rem(ug, chunks_per_t)
            return out_hbm.at[t, :, pl.ds(cb * _CB, _CB)]

        def fire_gather(b, u):
            pltpu.async_copy(scr_hbm.at[idx_v.at[u]], rows[b], gsems[b])

        def wait_gather(b):
            pltpu.make_async_copy(
                scr_hbm.at[pl.ds(0, _CB)], rows[b], gsems[b]).wait()

        def transpose(src, dst):
            @plsc.parallel_loop(0, _CB // _L, unroll=2)
            def bblk(bg):
                b0 = bg * _L
                bvec = rd + b0
                for cg in range(d // _L):
                    for s in range(_L):
                        cvec = cds[s] + (cg * _L)
                        val = plsc.load_gather(src, [bvec, cvec])
                        plsc.store_scatter(dst, [cvec, bvec], val)

        def fire_store(b, u):
            pltpu.make_async_copy(tbs[b], out_slice(u), ssems[b]).start()

        def wait_store(b, u):
            pltpu.make_async_copy(tbs[b], out_slice(u), ssems[b]).wait()

        fire_gather(0, 0)
        fire_gather(1, 1)

        def body(p, carry):
            for b in range(2):
                u = 2 * p + b
                wait_gather(b)

                @pl.when(p > 0)
                def _():
                    wait_store(b, u)

                transpose(rows[b], tbs[b])
                fire_store(b, u)

                @pl.when(u + 2 < chunks_per_w)
                def _():
                    fire_gather(b, u + 2)

            return carry

        lax.fori_loop(0, pairs, body, 0)
        wait_store(0, chunks_per_w - 2)
        wait_store(1, chunks_per_w - 1)

    return k2


def kernel(input, embedding_weight):
    b, h = input.shape
    v, d = embedding_weight.shape
    v_pad = ((v + _VB - 1) // _VB) * _VB
    wt = embedding_weight.T
    ntail = v - (v // _VB) * _VB
    tail = jnp.pad(embedding_weight[v - ntail:].T, ((0, 0), (0, _VB - ntail)))
    scratch = _transpose_call(v, d)(wt, tail)
    idx2d = input.T.reshape((b * h) // _CB, _CB)
    out3 = _gather_call(h, b, v_pad, d)(idx2d, scratch)
    return out3.transpose(2, 0, 1)


# speedup vs baseline: 1.0797x; 1.0797x over previous
import functools

import jax
import jax.numpy as jnp
from jax import lax
from jax.experimental import pallas as pl
from jax.experimental.pallas import tpu as pltpu
from jax.experimental.pallas import tpu_sc as plsc

_NC = 2
_NS = 16
_NW = _NC * _NS
_CB = 128
_K = 2
_SB = _K * _CB
_L = 16


@functools.lru_cache(maxsize=None)
def _gather_call(t_dim, b_dim, d):
    sc_per_t = b_dim // _SB
    sc_total = t_dim * sc_per_t
    sc_per_w = sc_total // _NW
    pairs = sc_per_w // 2
    mesh = plsc.VectorSubcoreMesh(core_axis_name="c", subcore_axis_name="s")

    @functools.partial(
        pl.kernel,
        mesh=mesh,
        out_type=jax.ShapeDtypeStruct((t_dim, d, b_dim), jnp.float32),
        compiler_params=pltpu.CompilerParams(
            use_tc_tiling_on_sc=False, needs_layout_passes=False),
        scratch_types=[
            pltpu.VMEM((sc_per_w * _K, _CB), jnp.int32),
            pltpu.VMEM((_SB, d), jnp.float32),
            pltpu.VMEM((_SB, d), jnp.float32),
            pltpu.VMEM((d, _SB + 1), jnp.float32),
            pltpu.VMEM((d, _SB + 1), jnp.float32),
            pltpu.SemaphoreType.DMA,
            pltpu.SemaphoreType.DMA,
            pltpu.SemaphoreType.DMA,
            pltpu.SemaphoreType.DMA,
        ],
    )
    def k(idx_hbm, table_hbm, out_hbm, idx_v, rows0, rows1, tb0, tb1,
          gsem0, gsem1, ssem0, ssem1):
        wid = lax.axis_index("s") * _NC + lax.axis_index("c")
        c0 = wid * sc_per_w * _K
        pltpu.sync_copy(idx_hbm.at[pl.ds(c0, sc_per_w * _K)], idx_v)
        rows = (rows0, rows1)
        tbs = (tb0, tb1)
        gsems = (gsem0, gsem1)
        ssems = (ssem0, ssem1)
        col_ids = [lax.iota(jnp.int32, _L) + (_L * g) for g in range(d // _L)]

        def out_slice(u):
            ug = c0 // _K + u
            t = ug // sc_per_t
            sb = lax.rem(ug, sc_per_t)
            return out_hbm.at[t, :, pl.ds(sb * _SB, _SB)]

        def fire_gathers(b, u):
            for kk in range(_K):
                pltpu.async_copy(
                    table_hbm.at[idx_v.at[u * _K + kk]],
                    rows[b].at[pl.ds(kk * _CB, _CB)],
                    gsems[b])

        def wait_gathers(b, u):
            pltpu.make_async_copy(
                table_hbm.at[pl.ds(0, _SB)], rows[b], gsems[b]).wait()

        def transpose(b):
            src, dst = rows[b], tbs[b]

            @plsc.parallel_loop(0, _SB, unroll=4)
            def rowfn(r):
                r_ids = jnp.full((_L,), r, jnp.int32)
                for g in range(d // _L):
                    v = src[r, pl.ds(_L * g, _L)]
                    plsc.store_scatter(dst, [col_ids[g], r_ids], v)

        def fire_store(b, u):
            pltpu.make_async_copy(
                tbs[b].at[:, pl.ds(0, _SB)], out_slice(u), ssems[b]).start()

        def wait_store(b, u):
            pltpu.make_async_copy(
                tbs[b].at[:, pl.ds(0, _SB)], out_slice(u), ssems[b]).wait()

        fire_gathers(0, 0)
        fire_gathers(1, 1)

        def body(p, carry):
            for b in range(2):
                u = 2 * p + b
                wait_gathers(b, u)

                @pl.when(p > 0)
                def _():
                    wait_store(b, u)

                transpose(b)
                fire_store(b, u)

                @pl.when(u + 2 < sc_per_w)
                def _():
                    fire_gathers(b, u + 2)

            return carry

        lax.fori_loop(0, pairs, body, 0)
        wait_store(0, sc_per_w - 2)
        wait_store(1, sc_per_w - 1)

    return k


def kernel(input, embedding_weight):
    b, h = input.shape
    v, d = embedding_weight.shape
    idx2d = input.T.reshape((b * h) // _CB, _CB)
    out3 = _gather_call(h, b, d)(idx2d, embedding_weight)
    return out3.transpose(2, 0, 1)
